# Initial kernel scaffold; baseline (speedup 1.0000x reference)
import jax
import jax.numpy as jnp
from jax.experimental import pallas as pl


def kernel(x, W1a, b1a, W1b, b1b, W2a, b2a, W2b, b2b):
    raise NotImplementedError("write your pallas kernel here")


# speedup vs baseline: 3.3839x; 3.3839x over previous
import functools

import jax
import jax.numpy as jnp
from jax.experimental import pallas as pl
from jax.experimental.pallas import tpu as pltpu

N = 10000
NP = 10240
BM = 256
NB = NP // BM

_PREC = jax.lax.Precision.DEFAULT


def _sq_kernel(y_ref, sq_ref):
    y = y_ref[...]
    sq_ref[...] = jnp.sum(y * y, axis=1, keepdims=True)


def _knn_kernel(y_ref, sqc_ref, sqr_ref, idx_ref, dist_ref, *, k):
    i = pl.program_id(0)
    y = y_ref[...]
    yi = y_ref[pl.ds(i * BM, BM), :]
    dot = jax.lax.dot_general(yi, y, (((1,), (1,)), ((), ())),
                              preferred_element_type=jnp.float32,
                              precision=_PREC)
    dist = (sqc_ref[...] - 2.0 * dot) + sqr_ref[...]
    col = jax.lax.broadcasted_iota(jnp.int32, (BM, NP), 1)
    dist = jnp.where(col >= N, jnp.inf, dist)
    dist_ref[...] = dist
    for e in range(k):
        dcur = dist_ref[...]
        v = jnp.min(dcur, axis=1, keepdims=True)
        idx_e = jnp.min(jnp.where(dcur == v, col, NP), axis=1,
                        keepdims=True).astype(jnp.int32)
        idx_ref[:, pl.ds(e, 1)] = idx_e
        dist_ref[...] = jnp.where(col == idx_e, jnp.inf, dcur)


def _edge_mlp_kernel(idx_ref, y_ref, wa_ref, ba_ref, wb_ref, bb_ref, out_ref,
                     g_ref, *, k, d):
    i = pl.program_id(0)
    xi = y_ref[pl.ds(i * BM, BM), :]
    wa = wa_ref[...]
    wb = wb_ref[...]
    ba = ba_ref[...]
    acc = None
    for e in range(k):
        def body(r, _):
            j = idx_ref[r, e]
            g_ref[pl.ds(r, 1), :] = y_ref[pl.ds(j, 1), :]
            return 0
        jax.lax.fori_loop(0, BM, body, 0)
        feat = jnp.concatenate([xi, g_ref[...] - xi], axis=1)
        h = jnp.maximum(jnp.dot(feat, wa, preferred_element_type=jnp.float32,
                                precision=_PREC) + ba, 0.0)
        p = jnp.dot(h, wb, preferred_element_type=jnp.float32, precision=_PREC)
        acc = p if acc is None else jnp.maximum(acc, p)
    out_ref[...] = acc + bb_ref[...]


def _knn(y_pad, k, d):
    sq = pl.pallas_call(
        _sq_kernel,
        grid=(NB,),
        in_specs=[pl.BlockSpec((BM, d), lambda i: (i, 0))],
        out_specs=pl.BlockSpec((BM, 1), lambda i: (i, 0)),
        out_shape=jax.ShapeDtypeStruct((NP, 1), jnp.float32),
    )(y_pad)
    sq_row = sq.reshape(1, NP)
    idx = pl.pallas_call(
        functools.partial(_knn_kernel, k=k),
        grid=(NB,),
        in_specs=[
            pl.BlockSpec((NP, d), lambda i: (0, 0)),
            pl.BlockSpec((BM, 1), lambda i: (i, 0)),
            pl.BlockSpec((1, NP), lambda i: (0, 0)),
        ],
        out_specs=pl.BlockSpec((BM, k), lambda i: (i, 0)),
        out_shape=jax.ShapeDtypeStruct((NP, k), jnp.int32),
        scratch_shapes=[pltpu.VMEM((BM, NP), jnp.float32)],
    )(y_pad, sq, sq_row)
    return idx


def _edge(y_pad, idx, wa, ba, wb, bb, k, d, dh, dout):
    return pl.pallas_call(
        functools.partial(_edge_mlp_kernel, k=k, d=d),
        grid=(NB,),
        in_specs=[
            pl.BlockSpec((BM, k), lambda i: (i, 0),
                         memory_space=pltpu.MemorySpace.SMEM),
            pl.BlockSpec((NP, d), lambda i: (0, 0)),
            pl.BlockSpec((2 * d, dh), lambda i: (0, 0)),
            pl.BlockSpec((1, dh), lambda i: (0, 0)),
            pl.BlockSpec((dh, dout), lambda i: (0, 0)),
            pl.BlockSpec((1, dout), lambda i: (0, 0)),
        ],
        out_specs=pl.BlockSpec((BM, dout), lambda i: (i, 0)),
        out_shape=jax.ShapeDtypeStruct((NP, dout), jnp.float32),
        scratch_shapes=[pltpu.VMEM((BM, d), jnp.float32)],
    )(idx, y_pad, wa, ba, wb, bb)


def _layer(y_pad, wa, ba, wb, bb, k, d, dh, dout):
    idx = _knn(y_pad, k, d)
    return _edge(y_pad, idx, wa, ba, wb, bb, k, d, dh, dout)


@jax.jit
def kernel(x, W1a, b1a, W1b, b1b, W2a, b2a, W2b, b2b):
    x_pad = jnp.pad(x, ((0, NP - N), (0, 0)))
    h = _layer(x_pad, W1a, b1a.reshape(1, -1), W1b, b1b.reshape(1, -1),
               k=16, d=128, dh=256, dout=256)
    out = _layer(h, W2a, b2a.reshape(1, -1), W2b, b2b.reshape(1, -1),
                 k=8, d=256, dh=256, dout=256)
    return out[:N]


# speedup vs baseline: 5.2124x; 1.5403x over previous
import functools

import jax
import jax.numpy as jnp
from jax.experimental import pallas as pl
from jax.experimental.pallas import tpu as pltpu

N = 10000
NP = 10240
BM = 256
NB = NP // BM
WC = 128
C = NP // WC
T = 4
CT = C * T

_PREC = jax.lax.Precision.DEFAULT


def _sq_kernel(y_ref, sq_ref):
    y = y_ref[...]
    sq_ref[...] = jnp.sum(y * y, axis=1, keepdims=True)


def _knn_kernel(y_ref, sq_ref, sqr_ref, idxT_ref, dist_ref, mv_ref, gi_ref,
                *, k):
    i = pl.program_id(0)
    y = y_ref[...]
    yi = y_ref[pl.ds(i * BM, BM), :]
    dotT = jax.lax.dot_general(y, yi, (((1,), (1,)), ((), ())),
                               preferred_element_type=jnp.float32,
                               precision=_PREC)
    sqi_row = sqr_ref[:, pl.ds(i * BM, BM)]
    dist = (sqi_row - 2.0 * dotT) + sq_ref[...]
    rowg_full = jax.lax.broadcasted_iota(jnp.int32, (NP, BM), 0)
    dist_ref[...] = jnp.where(rowg_full >= N, jnp.inf, dist)

    def phase_a(c, _):
        base = c * WC
        ch = dist_ref[pl.ds(base, WC), :]
        rowg = jax.lax.broadcasted_iota(jnp.int32, (WC, BM), 0) + base
        for t in range(T):
            m = jnp.min(ch, axis=0, keepdims=True)
            g = jnp.min(jnp.where(ch == m, rowg, NP), axis=0,
                        keepdims=True)
            mv_ref[pl.ds(c * T + t, 1), :] = m
            gi_ref[pl.ds(c * T + t, 1), :] = g
            if t < T - 1:
                ch = jnp.where(rowg == g, jnp.inf, ch)
        return 0
    jax.lax.fori_loop(0, C, phase_a, 0)

    slot = jax.lax.broadcasted_iota(jnp.int32, (CT, BM), 0)
    for e in range(k):
        mv = mv_ref[...]
        v = jnp.min(mv, axis=0, keepdims=True)
        p = jnp.min(jnp.where(mv == v, slot, CT), axis=0,
                    keepdims=True)
        idx_e = jnp.min(jnp.where(slot == p, gi_ref[...], NP), axis=0,
                        keepdims=True)
        idxT_ref[pl.ds(e, 1), :] = idx_e
        mv_ref[...] = jnp.where(slot == p, jnp.inf, mv)
        if e < k - 1:
            need = (p & (T - 1)) == (T - 1)
            c_p = p >> 2
            any_need = jnp.any(need)

            @pl.when(any_need)
            def _repair(v=v, idx_e=idx_e, need=need, c_p=c_p):
                def rep(c, _):
                    base = c * WC
                    ch = dist_ref[pl.ds(base, WC), :]
                    rowg = jax.lax.broadcasted_iota(
                        jnp.int32, (WC, BM), 0) + base
                    elig = (ch > v) | ((ch == v) & (rowg > idx_e))
                    chm = jnp.where(elig, ch, jnp.inf)
                    sel = need & (c_p == c)
                    for t in range(T):
                        m = jnp.min(chm, axis=0, keepdims=True)
                        g = jnp.min(jnp.where(chm == m, rowg, NP), axis=0,
                                    keepdims=True)
                        s = c * T + t
                        mv_ref[pl.ds(s, 1), :] = jnp.where(
                            sel, m, mv_ref[pl.ds(s, 1), :])
                        gi_ref[pl.ds(s, 1), :] = jnp.where(
                            sel, g, gi_ref[pl.ds(s, 1), :])
                        if t < T - 1:
                            chm = jnp.where(rowg == g, jnp.inf, chm)
                    return 0
                jax.lax.fori_loop(0, C, rep, 0)


def _edge_mlp_kernel(idxT_ref, y_ref, wa_ref, ba_ref, wb_ref, bb_ref, out_ref,
                     ga_ref, gb_ref, *, k, d):
    i = pl.program_id(0)
    xi = y_ref[pl.ds(i * BM, BM), :]
    wa = wa_ref[...]
    wb = wb_ref[...]
    ba = ba_ref[...]
    acc = None
    bufs = (ga_ref, gb_ref)
    for e in range(k):
        gr = bufs[e & 1]

        def body(r8, _, gr=gr, e=e):
            b = r8 * 8
            for u in range(8):
                j = idxT_ref[e, b + u]
                gr[pl.ds(b + u, 1), :] = y_ref[pl.ds(j, 1), :]
            return 0
        jax.lax.fori_loop(0, BM // 8, body, 0)
        feat = jnp.concatenate([xi, gr[...] - xi], axis=1)
        h = jnp.maximum(jnp.dot(feat, wa, preferred_element_type=jnp.float32,
                                precision=_PREC) + ba, 0.0)
        pp = jnp.dot(h, wb, preferred_element_type=jnp.float32,
                     precision=_PREC)
        acc = pp if acc is None else jnp.maximum(acc, pp)
    out_ref[...] = acc + bb_ref[...]


def _knn(y_pad, k, d):
    sq = pl.pallas_call(
        _sq_kernel,
        grid=(NB,),
        in_specs=[pl.BlockSpec((BM, d), lambda i: (i, 0))],
        out_specs=pl.BlockSpec((BM, 1), lambda i: (i, 0)),
        out_shape=jax.ShapeDtypeStruct((NP, 1), jnp.float32),
    )(y_pad)
    sq_row = sq.reshape(1, NP)
    idxT = pl.pallas_call(
        functools.partial(_knn_kernel, k=k),
        grid=(NB,),
        in_specs=[
            pl.BlockSpec((NP, d), lambda i: (0, 0)),
            pl.BlockSpec((NP, 1), lambda i: (0, 0)),
            pl.BlockSpec((1, NP), lambda i: (0, 0)),
        ],
        out_specs=pl.BlockSpec((k, BM), lambda i: (0, i)),
        out_shape=jax.ShapeDtypeStruct((k, NP), jnp.int32),
        scratch_shapes=[pltpu.VMEM((NP, BM), jnp.float32),
                        pltpu.VMEM((CT, BM), jnp.float32),
                        pltpu.VMEM((CT, BM), jnp.int32)],
    )(y_pad, sq, sq_row)
    return idxT


def _edge(y_pad, idxT, wa, ba, wb, bb, k, d, dh, dout):
    return pl.pallas_call(
        functools.partial(_edge_mlp_kernel, k=k, d=d),
        grid=(NB,),
        in_specs=[
            pl.BlockSpec((k, BM), lambda i: (0, i),
                         memory_space=pltpu.MemorySpace.SMEM),
            pl.BlockSpec((NP, d), lambda i: (0, 0)),
            pl.BlockSpec((2 * d, dh), lambda i: (0, 0)),
            pl.BlockSpec((1, dh), lambda i: (0, 0)),
            pl.BlockSpec((dh, dout), lambda i: (0, 0)),
            pl.BlockSpec((1, dout), lambda i: (0, 0)),
        ],
        out_specs=pl.BlockSpec((BM, dout), lambda i: (i, 0)),
        out_shape=jax.ShapeDtypeStruct((NP, dout), jnp.float32),
        scratch_shapes=[pltpu.VMEM((BM, d), jnp.float32),
                        pltpu.VMEM((BM, d), jnp.float32)],
    )(idxT, y_pad, wa, ba, wb, bb)


def _layer(y_pad, wa, ba, wb, bb, k, d, dh, dout):
    idxT = _knn(y_pad, k, d)
    return _edge(y_pad, idxT, wa, ba, wb, bb, k, d, dh, dout)


@jax.jit
def kernel(x, W1a, b1a, W1b, b1b, W2a, b2a, W2b, b2b):
    x_pad = jnp.pad(x, ((0, NP - N), (0, 0)))
    h = _layer(x_pad, W1a, b1a.reshape(1, -1), W1b, b1b.reshape(1, -1),
               k=16, d=128, dh=256, dout=256)
    out = _layer(h, W2a, b2a.reshape(1, -1), W2b, b2b.reshape(1, -1),
                 k=8, d=256, dh=256, dout=256)
    return out[:N]


# speedup vs baseline: 5.7587x; 1.1048x over previous
import functools

import jax
import jax.numpy as jnp
from jax.experimental import pallas as pl
from jax.experimental.pallas import tpu as pltpu

N = 10000
NP = 10240
BM = 512
NB = NP // BM
WC = 128
C = NP // WC
T = 4
CT = C * T

_PREC = jax.lax.Precision.DEFAULT


def _sq_kernel(y_ref, sq_ref):
    y = y_ref[...]
    sq_ref[...] = jnp.sum(y * y, axis=1, keepdims=True)


def _knn_kernel(y_ref, sq_ref, sqr_ref, idxT_ref, dist_ref, mv_ref, gi_ref,
                *, k):
    i = pl.program_id(0)
    y = y_ref[...]
    yi = y_ref[pl.ds(i * BM, BM), :]
    dotT = jax.lax.dot_general(y, yi, (((1,), (1,)), ((), ())),
                               preferred_element_type=jnp.float32,
                               precision=_PREC)
    sqi_row = sqr_ref[:, pl.ds(i * BM, BM)]
    dist = (sqi_row - 2.0 * dotT) + sq_ref[...]
    rowg_full = jax.lax.broadcasted_iota(jnp.int32, (NP, BM), 0)
    dist_ref[...] = jnp.where(rowg_full >= N, jnp.inf, dist)

    def phase_a(c, _):
        base = c * WC
        ch = dist_ref[pl.ds(base, WC), :]
        rowg = jax.lax.broadcasted_iota(jnp.int32, (WC, BM), 0) + base
        for t in range(T):
            m = jnp.min(ch, axis=0, keepdims=True)
            g = jnp.min(jnp.where(ch == m, rowg, NP), axis=0,
                        keepdims=True)
            mv_ref[pl.ds(c * T + t, 1), :] = m
            gi_ref[pl.ds(c * T + t, 1), :] = g
            if t < T - 1:
                ch = jnp.where(rowg == g, jnp.inf, ch)
        return 0
    jax.lax.fori_loop(0, C, phase_a, 0)

    slot = jax.lax.broadcasted_iota(jnp.int32, (CT, BM), 0)
    for e in range(k):
        mv = mv_ref[...]
        v = jnp.min(mv, axis=0, keepdims=True)
        p = jnp.min(jnp.where(mv == v, slot, CT), axis=0,
                    keepdims=True)
        idx_e = jnp.min(jnp.where(slot == p, gi_ref[...], NP), axis=0,
                        keepdims=True)
        idxT_ref[pl.ds(e, 1), :] = idx_e
        mv_ref[...] = jnp.where(slot == p, jnp.inf, mv)
        if e < k - 1:
            need = (p & (T - 1)) == (T - 1)
            c_p = p >> 2
            any_need = jnp.any(need)

            @pl.when(any_need)
            def _repair(v=v, idx_e=idx_e, need=need, c_p=c_p):
                def rep(c, _):
                    base = c * WC
                    ch = dist_ref[pl.ds(base, WC), :]
                    rowg = jax.lax.broadcasted_iota(
                        jnp.int32, (WC, BM), 0) + base
                    elig = (ch > v) | ((ch == v) & (rowg > idx_e))
                    chm = jnp.where(elig, ch, jnp.inf)
                    sel = need & (c_p == c)
                    for t in range(T):
                        m = jnp.min(chm, axis=0, keepdims=True)
                        g = jnp.min(jnp.where(chm == m, rowg, NP), axis=0,
                                    keepdims=True)
                        s = c * T + t
                        mv_ref[pl.ds(s, 1), :] = jnp.where(
                            sel, m, mv_ref[pl.ds(s, 1), :])
                        gi_ref[pl.ds(s, 1), :] = jnp.where(
                            sel, g, gi_ref[pl.ds(s, 1), :])
                        if t < T - 1:
                            chm = jnp.where(rowg == g, jnp.inf, chm)
                    return 0
                jax.lax.fori_loop(0, C, rep, 0)


def _edge_mlp_kernel(idxT_ref, y_ref, wa_ref, ba_ref, wb_ref, bb_ref, out_ref,
                     ga_ref, gb_ref, *, k, d):
    i = pl.program_id(0)
    xi = y_ref[pl.ds(i * BM, BM), :]
    wa = wa_ref[...]
    wb = wb_ref[...]
    ba = ba_ref[...]
    acc = None
    bufs = (ga_ref, gb_ref)
    for e in range(k):
        gr = bufs[e & 1]

        def body(r8, _, gr=gr, e=e):
            b = r8 * 8
            for u in range(8):
                j = idxT_ref[e, b + u]
                gr[pl.ds(b + u, 1), :] = y_ref[pl.ds(j, 1), :]
            return 0
        jax.lax.fori_loop(0, BM // 8, body, 0)
        feat = jnp.concatenate([xi, gr[...] - xi], axis=1)
        h = jnp.maximum(jnp.dot(feat, wa, preferred_element_type=jnp.float32,
                                precision=_PREC) + ba, 0.0)
        pp = jnp.dot(h, wb, preferred_element_type=jnp.float32,
                     precision=_PREC)
        acc = pp if acc is None else jnp.maximum(acc, pp)
    out_ref[...] = acc + bb_ref[...]


def _knn(y_pad, k, d):
    sq = pl.pallas_call(
        _sq_kernel,
        grid=(NB,),
        in_specs=[pl.BlockSpec((BM, d), lambda i: (i, 0))],
        out_specs=pl.BlockSpec((BM, 1), lambda i: (i, 0)),
        out_shape=jax.ShapeDtypeStruct((NP, 1), jnp.float32),
    )(y_pad)
    sq_row = sq.reshape(1, NP)
    idxT = pl.pallas_call(
        functools.partial(_knn_kernel, k=k),
        grid=(NB,),
        in_specs=[
            pl.BlockSpec((NP, d), lambda i: (0, 0)),
            pl.BlockSpec((NP, 1), lambda i: (0, 0)),
            pl.BlockSpec((1, NP), lambda i: (0, 0)),
        ],
        out_specs=pl.BlockSpec((k, BM), lambda i: (0, i)),
        out_shape=jax.ShapeDtypeStruct((k, NP), jnp.int32),
        scratch_shapes=[pltpu.VMEM((NP, BM), jnp.float32),
                        pltpu.VMEM((CT, BM), jnp.float32),
                        pltpu.VMEM((CT, BM), jnp.int32)],
    )(y_pad, sq, sq_row)
    return idxT


def _edge(y_pad, idxT, wa, ba, wb, bb, k, d, dh, dout):
    return pl.pallas_call(
        functools.partial(_edge_mlp_kernel, k=k, d=d),
        grid=(NB,),
        in_specs=[
            pl.BlockSpec((k, BM), lambda i: (0, i),
                         memory_space=pltpu.MemorySpace.SMEM),
            pl.BlockSpec((NP, d), lambda i: (0, 0)),
            pl.BlockSpec((2 * d, dh), lambda i: (0, 0)),
            pl.BlockSpec((1, dh), lambda i: (0, 0)),
            pl.BlockSpec((dh, dout), lambda i: (0, 0)),
            pl.BlockSpec((1, dout), lambda i: (0, 0)),
        ],
        out_specs=pl.BlockSpec((BM, dout), lambda i: (i, 0)),
        out_shape=jax.ShapeDtypeStruct((NP, dout), jnp.float32),
        scratch_shapes=[pltpu.VMEM((BM, d), jnp.float32),
                        pltpu.VMEM((BM, d), jnp.float32)],
    )(idxT, y_pad, wa, ba, wb, bb)


def _layer(y_pad, wa, ba, wb, bb, k, d, dh, dout):
    idxT = _knn(y_pad, k, d)
    return _edge(y_pad, idxT, wa, ba, wb, bb, k, d, dh, dout)


@jax.jit
def kernel(x, W1a, b1a, W1b, b1b, W2a, b2a, W2b, b2b):
    x_pad = jnp.pad(x, ((0, NP - N), (0, 0)))
    h = _layer(x_pad, W1a, b1a.reshape(1, -1), W1b, b1b.reshape(1, -1),
               k=16, d=128, dh=256, dout=256)
    out = _layer(h, W2a, b2a.reshape(1, -1), W2b, b2b.reshape(1, -1),
                 k=8, d=256, dh=256, dout=256)
    return out[:N]


# speedup vs baseline: 7.0235x; 1.2196x over previous
import functools

import jax
import jax.numpy as jnp
from jax.experimental import pallas as pl
from jax.experimental.pallas import tpu as pltpu
from jax.experimental.pallas import tpu_sc as plsc

N = 10000
NP = 10240
BM = 512
NB = NP // BM
WC = 128
C = NP // WC
T = 4
CT = C * T

_PREC = jax.lax.Precision.DEFAULT


def _sq_kernel(y_ref, sq_ref):
    y = y_ref[...]
    sq_ref[...] = jnp.sum(y * y, axis=1, keepdims=True)


def _knn_kernel(y_ref, sq_ref, sqr_ref, idxT_ref, dist_ref, mv_ref, gi_ref,
                *, k):
    i = pl.program_id(0)
    y = y_ref[...]
    yi = y_ref[pl.ds(i * BM, BM), :]
    dotT = jax.lax.dot_general(y, yi, (((1,), (1,)), ((), ())),
                               preferred_element_type=jnp.float32,
                               precision=_PREC)
    sqi_row = sqr_ref[:, pl.ds(i * BM, BM)]
    dist = (sqi_row - 2.0 * dotT) + sq_ref[...]
    rowg_full = jax.lax.broadcasted_iota(jnp.int32, (NP, BM), 0)
    dist_ref[...] = jnp.where(rowg_full >= N, jnp.inf, dist)

    def phase_a(c, _):
        base = c * WC
        ch = dist_ref[pl.ds(base, WC), :]
        rowg = jax.lax.broadcasted_iota(jnp.int32, (WC, BM), 0) + base
        for t in range(T):
            m = jnp.min(ch, axis=0, keepdims=True)
            g = jnp.min(jnp.where(ch == m, rowg, NP), axis=0,
                        keepdims=True)
            mv_ref[pl.ds(c * T + t, 1), :] = m
            gi_ref[pl.ds(c * T + t, 1), :] = g
            if t < T - 1:
                ch = jnp.where(rowg == g, jnp.inf, ch)
        return 0
    jax.lax.fori_loop(0, C, phase_a, 0)

    slot = jax.lax.broadcasted_iota(jnp.int32, (CT, BM), 0)
    for e in range(k):
        mv = mv_ref[...]
        v = jnp.min(mv, axis=0, keepdims=True)
        p = jnp.min(jnp.where(mv == v, slot, CT), axis=0,
                    keepdims=True)
        idx_e = jnp.min(jnp.where(slot == p, gi_ref[...], NP), axis=0,
                        keepdims=True)
        idxT_ref[pl.ds(e, 1), :] = idx_e
        mv_ref[...] = jnp.where(slot == p, jnp.inf, mv)
        if e < k - 1:
            need = (p & (T - 1)) == (T - 1)
            c_p = p >> 2
            any_need = jnp.any(need)

            @pl.when(any_need)
            def _repair(v=v, idx_e=idx_e, need=need, c_p=c_p):
                def rep(c, _):
                    base = c * WC
                    ch = dist_ref[pl.ds(base, WC), :]
                    rowg = jax.lax.broadcasted_iota(
                        jnp.int32, (WC, BM), 0) + base
                    elig = (ch > v) | ((ch == v) & (rowg > idx_e))
                    chm = jnp.where(elig, ch, jnp.inf)
                    sel = need & (c_p == c)
                    for t in range(T):
                        m = jnp.min(chm, axis=0, keepdims=True)
                        g = jnp.min(jnp.where(chm == m, rowg, NP), axis=0,
                                    keepdims=True)
                        s = c * T + t
                        mv_ref[pl.ds(s, 1), :] = jnp.where(
                            sel, m, mv_ref[pl.ds(s, 1), :])
                        gi_ref[pl.ds(s, 1), :] = jnp.where(
                            sel, g, gi_ref[pl.ds(s, 1), :])
                        if t < T - 1:
                            chm = jnp.where(rowg == g, jnp.inf, chm)
                    return 0
                jax.lax.fori_loop(0, C, rep, 0)


_GW = 128


def _sc_gather(y_pad, idx_flat, k, d):
    ni = NP * k
    mesh = plsc.VectorSubcoreMesh(core_axis_name="core",
                                  subcore_axis_name="subcore")

    @pl.kernel(out_type=jax.ShapeDtypeStruct((ni, d), jnp.float32), mesh=mesh)
    def gk(y_hbm, i_hbm, o_hbm):
        def body(i_vmem, o_vmem):
            pltpu.sync_copy(y_hbm.at[i_vmem.at[0]], o_vmem)

        pltpu.emit_pipeline(
            body,
            grid=(ni // _GW,),
            in_specs=[pl.BlockSpec((1, _GW), index_map=lambda i: (0, i))],
            out_specs=[pl.BlockSpec((_GW, d), index_map=lambda i: (i, 0))],
            core_axis_name=("core", "subcore"),
            dimension_semantics=(pltpu.PARALLEL,),
        )(i_hbm, o_hbm)

    return gk(y_pad, idx_flat)


def _edge_mlp_g_kernel(g_ref, y_ref, wa_ref, ba_ref, wb_ref, bb_ref, out_ref,
                       *, k, d):
    i = pl.program_id(0)
    xi = y_ref[pl.ds(i * BM, BM), :]
    wa = wa_ref[...]
    wb = wb_ref[...]
    ba = ba_ref[...]
    acc = None
    for e in range(k):
        xj = g_ref[:, pl.ds(e * d, d)]
        feat = jnp.concatenate([xi, xj - xi], axis=1)
        h = jnp.maximum(jnp.dot(feat, wa, preferred_element_type=jnp.float32,
                                precision=_PREC) + ba, 0.0)
        pp = jnp.dot(h, wb, preferred_element_type=jnp.float32,
                     precision=_PREC)
        acc = pp if acc is None else jnp.maximum(acc, pp)
    out_ref[...] = acc + bb_ref[...]


def _edge_g(y_pad, g, wa, ba, wb, bb, k, d, dh, dout):
    return pl.pallas_call(
        functools.partial(_edge_mlp_g_kernel, k=k, d=d),
        grid=(NB,),
        in_specs=[
            pl.BlockSpec((BM, k * d), lambda i: (i, 0)),
            pl.BlockSpec((NP, d), lambda i: (0, 0)),
            pl.BlockSpec((2 * d, dh), lambda i: (0, 0)),
            pl.BlockSpec((1, dh), lambda i: (0, 0)),
            pl.BlockSpec((dh, dout), lambda i: (0, 0)),
            pl.BlockSpec((1, dout), lambda i: (0, 0)),
        ],
        out_specs=pl.BlockSpec((BM, dout), lambda i: (i, 0)),
        out_shape=jax.ShapeDtypeStruct((NP, dout), jnp.float32),
    )(g, y_pad, wa, ba, wb, bb)


def _edge_mlp_kernel(idxT_ref, y_ref, wa_ref, ba_ref, wb_ref, bb_ref, out_ref,
                     ga_ref, gb_ref, *, k, d):
    i = pl.program_id(0)
    xi = y_ref[pl.ds(i * BM, BM), :]
    wa = wa_ref[...]
    wb = wb_ref[...]
    ba = ba_ref[...]
    acc = None
    bufs = (ga_ref, gb_ref)
    for e in range(k):
        gr = bufs[e & 1]

        def body(r8, _, gr=gr, e=e):
            b = r8 * 8
            for u in range(8):
                j = idxT_ref[e, b + u]
                gr[pl.ds(b + u, 1), :] = y_ref[pl.ds(j, 1), :]
            return 0
        jax.lax.fori_loop(0, BM // 8, body, 0)
        feat = jnp.concatenate([xi, gr[...] - xi], axis=1)
        h = jnp.maximum(jnp.dot(feat, wa, preferred_element_type=jnp.float32,
                                precision=_PREC) + ba, 0.0)
        pp = jnp.dot(h, wb, preferred_element_type=jnp.float32,
                     precision=_PREC)
        acc = pp if acc is None else jnp.maximum(acc, pp)
    out_ref[...] = acc + bb_ref[...]


def _knn(y_pad, k, d):
    sq = pl.pallas_call(
        _sq_kernel,
        grid=(NB,),
        in_specs=[pl.BlockSpec((BM, d), lambda i: (i, 0))],
        out_specs=pl.BlockSpec((BM, 1), lambda i: (i, 0)),
        out_shape=jax.ShapeDtypeStruct((NP, 1), jnp.float32),
    )(y_pad)
    sq_row = sq.reshape(1, NP)
    idxT = pl.pallas_call(
        functools.partial(_knn_kernel, k=k),
        grid=(NB,),
        in_specs=[
            pl.BlockSpec((NP, d), lambda i: (0, 0)),
            pl.BlockSpec((NP, 1), lambda i: (0, 0)),
            pl.BlockSpec((1, NP), lambda i: (0, 0)),
        ],
        out_specs=pl.BlockSpec((k, BM), lambda i: (0, i)),
        out_shape=jax.ShapeDtypeStruct((k, NP), jnp.int32),
        scratch_shapes=[pltpu.VMEM((NP, BM), jnp.float32),
                        pltpu.VMEM((CT, BM), jnp.float32),
                        pltpu.VMEM((CT, BM), jnp.int32)],
    )(y_pad, sq, sq_row)
    return idxT


def _edge(y_pad, idxT, wa, ba, wb, bb, k, d, dh, dout):
    return pl.pallas_call(
        functools.partial(_edge_mlp_kernel, k=k, d=d),
        grid=(NB,),
        in_specs=[
            pl.BlockSpec((k, BM), lambda i: (0, i),
                         memory_space=pltpu.MemorySpace.SMEM),
            pl.BlockSpec((NP, d), lambda i: (0, 0)),
            pl.BlockSpec((2 * d, dh), lambda i: (0, 0)),
            pl.BlockSpec((1, dh), lambda i: (0, 0)),
            pl.BlockSpec((dh, dout), lambda i: (0, 0)),
            pl.BlockSpec((1, dout), lambda i: (0, 0)),
        ],
        out_specs=pl.BlockSpec((BM, dout), lambda i: (i, 0)),
        out_shape=jax.ShapeDtypeStruct((NP, dout), jnp.float32),
        scratch_shapes=[pltpu.VMEM((BM, d), jnp.float32),
                        pltpu.VMEM((BM, d), jnp.float32)],
    )(idxT, y_pad, wa, ba, wb, bb)


def _layer(y_pad, wa, ba, wb, bb, k, d, dh, dout):
    idxT = _knn(y_pad, k, d)
    idx_flat = idxT.T.reshape(1, NP * k)
    g = _sc_gather(y_pad, idx_flat, k, d).reshape(NP, k * d)
    return _edge_g(y_pad, g, wa, ba, wb, bb, k, d, dh, dout)


@jax.jit
def kernel(x, W1a, b1a, W1b, b1b, W2a, b2a, W2b, b2b):
    x_pad = jnp.pad(x, ((0, NP - N), (0, 0)))
    h = _layer(x_pad, W1a, b1a.reshape(1, -1), W1b, b1b.reshape(1, -1),
               k=16, d=128, dh=256, dout=256)
    out = _layer(h, W2a, b2a.reshape(1, -1), W2b, b2b.reshape(1, -1),
                 k=8, d=256, dh=256, dout=256)
    return out[:N]


# speedup vs baseline: 7.1474x; 1.0176x over previous
import functools

import jax
import jax.numpy as jnp
from jax.experimental import pallas as pl
from jax.experimental.pallas import tpu as pltpu
from jax.experimental.pallas import tpu_sc as plsc

N = 10000
NP = 10240
BM = 512
NB = NP // BM
WC = 128
C = NP // WC
T = 4
CT = C * T

_PREC = jax.lax.Precision.DEFAULT


def _sq_kernel(y_ref, sq_ref):
    y = y_ref[...]
    sq_ref[...] = jnp.sum(y * y, axis=1, keepdims=True)


def _knn_kernel(y_ref, sq_ref, sqr_ref, idxT_ref, dist_ref, mv_ref, gi_ref,
                *, k):
    i = pl.program_id(0)
    y = y_ref[...]
    yi = y_ref[pl.ds(i * BM, BM), :]
    dotT = jax.lax.dot_general(y, yi, (((1,), (1,)), ((), ())),
                               preferred_element_type=jnp.float32,
                               precision=_PREC)
    sqi_row = sqr_ref[:, pl.ds(i * BM, BM)]
    dist = (sqi_row - 2.0 * dotT) + sq_ref[...]
    rowg_full = jax.lax.broadcasted_iota(jnp.int32, (NP, BM), 0)
    dist_ref[...] = jnp.where(rowg_full >= N, jnp.inf, dist)

    def phase_a(c, _):
        base = c * WC
        ch = dist_ref[pl.ds(base, WC), :]
        rowg = jax.lax.broadcasted_iota(jnp.int32, (WC, BM), 0) + base
        for t in range(T):
            m = jnp.min(ch, axis=0, keepdims=True)
            g = jnp.min(jnp.where(ch == m, rowg, NP), axis=0,
                        keepdims=True)
            mv_ref[pl.ds(c * T + t, 1), :] = m
            gi_ref[pl.ds(c * T + t, 1), :] = g
            if t < T - 1:
                ch = jnp.where(rowg == g, jnp.inf, ch)
        return 0
    jax.lax.fori_loop(0, C, phase_a, 0)

    slot = jax.lax.broadcasted_iota(jnp.int32, (CT, BM), 0)
    for e in range(k):
        mv = mv_ref[...]
        v = jnp.min(mv, axis=0, keepdims=True)
        p = jnp.min(jnp.where(mv == v, slot, CT), axis=0,
                    keepdims=True)
        idx_e = jnp.min(jnp.where(slot == p, gi_ref[...], NP), axis=0,
                        keepdims=True)
        idxT_ref[pl.ds(e, 1), :] = idx_e
        mv_ref[...] = jnp.where(slot == p, jnp.inf, mv)
        if e < k - 1:
            need = (p & (T - 1)) == (T - 1)
            c_p = p >> 2
            any_need = jnp.any(need)

            @pl.when(any_need)
            def _repair(v=v, idx_e=idx_e, need=need, c_p=c_p):
                def rep(c, _):
                    base = c * WC
                    ch = dist_ref[pl.ds(base, WC), :]
                    rowg = jax.lax.broadcasted_iota(
                        jnp.int32, (WC, BM), 0) + base
                    elig = (ch > v) | ((ch == v) & (rowg > idx_e))
                    chm = jnp.where(elig, ch, jnp.inf)
                    sel = need & (c_p == c)
                    for t in range(T):
                        m = jnp.min(chm, axis=0, keepdims=True)
                        g = jnp.min(jnp.where(chm == m, rowg, NP), axis=0,
                                    keepdims=True)
                        s = c * T + t
                        mv_ref[pl.ds(s, 1), :] = jnp.where(
                            sel, m, mv_ref[pl.ds(s, 1), :])
                        gi_ref[pl.ds(s, 1), :] = jnp.where(
                            sel, g, gi_ref[pl.ds(s, 1), :])
                        if t < T - 1:
                            chm = jnp.where(rowg == g, jnp.inf, chm)
                    return 0
                jax.lax.fori_loop(0, C, rep, 0)


_GW = 128


def _sc_gather(y_pad, idx_flat, k, d):
    ni = idx_flat.shape[1]
    mesh = plsc.VectorSubcoreMesh(core_axis_name="core",
                                  subcore_axis_name="subcore")

    @pl.kernel(out_type=jax.ShapeDtypeStruct((ni, d), jnp.float32), mesh=mesh)
    def gk(y_hbm, i_hbm, o_hbm):
        def body(i_vmem, o_vmem):
            pltpu.sync_copy(y_hbm.at[i_vmem.at[0]], o_vmem)

        pltpu.emit_pipeline(
            body,
            grid=(ni // _GW,),
            in_specs=[pl.BlockSpec((1, _GW), index_map=lambda i: (0, i))],
            out_specs=[pl.BlockSpec((_GW, d), index_map=lambda i: (i, 0))],
            core_axis_name=("core", "subcore"),
            dimension_semantics=(pltpu.PARALLEL,),
        )(i_hbm, o_hbm)

    return gk(y_pad, idx_flat)


def _edge_mlp_g_kernel(g_ref, y_ref, wa_ref, ba_ref, wb_ref, bb_ref, out_ref,
                       *, k, d, base):
    i = pl.program_id(0)
    xi = y_ref[pl.ds(base + i * BM, BM), :]
    wa = wa_ref[...]
    wb = wb_ref[...]
    ba = ba_ref[...]
    acc = None
    for e in range(k):
        xj = g_ref[:, pl.ds(e * d, d)]
        feat = jnp.concatenate([xi, xj - xi], axis=1)
        h = jnp.maximum(jnp.dot(feat, wa, preferred_element_type=jnp.float32,
                                precision=_PREC) + ba, 0.0)
        pp = jnp.dot(h, wb, preferred_element_type=jnp.float32,
                     precision=_PREC)
        acc = pp if acc is None else jnp.maximum(acc, pp)
    out_ref[...] = acc + bb_ref[...]


def _edge_g(y_pad, g, wa, ba, wb, bb, k, d, dh, dout, base, nrows):
    return pl.pallas_call(
        functools.partial(_edge_mlp_g_kernel, k=k, d=d, base=base),
        grid=(nrows // BM,),
        in_specs=[
            pl.BlockSpec((BM, k * d), lambda i: (i, 0)),
            pl.BlockSpec((NP, d), lambda i: (0, 0)),
            pl.BlockSpec((2 * d, dh), lambda i: (0, 0)),
            pl.BlockSpec((1, dh), lambda i: (0, 0)),
            pl.BlockSpec((dh, dout), lambda i: (0, 0)),
            pl.BlockSpec((1, dout), lambda i: (0, 0)),
        ],
        out_specs=pl.BlockSpec((BM, dout), lambda i: (i, 0)),
        out_shape=jax.ShapeDtypeStruct((nrows, dout), jnp.float32),
    )(g, y_pad, wa, ba, wb, bb)


def _edge_mlp_kernel(idxT_ref, y_ref, wa_ref, ba_ref, wb_ref, bb_ref, out_ref,
                     ga_ref, gb_ref, *, k, d):
    i = pl.program_id(0)
    xi = y_ref[pl.ds(i * BM, BM), :]
    wa = wa_ref[...]
    wb = wb_ref[...]
    ba = ba_ref[...]
    acc = None
    bufs = (ga_ref, gb_ref)
    for e in range(k):
        gr = bufs[e & 1]

        def body(r8, _, gr=gr, e=e):
            b = r8 * 8
            for u in range(8):
                j = idxT_ref[e, b + u]
                gr[pl.ds(b + u, 1), :] = y_ref[pl.ds(j, 1), :]
            return 0
        jax.lax.fori_loop(0, BM // 8, body, 0)
        feat = jnp.concatenate([xi, gr[...] - xi], axis=1)
        h = jnp.maximum(jnp.dot(feat, wa, preferred_element_type=jnp.float32,
                                precision=_PREC) + ba, 0.0)
        pp = jnp.dot(h, wb, preferred_element_type=jnp.float32,
                     precision=_PREC)
        acc = pp if acc is None else jnp.maximum(acc, pp)
    out_ref[...] = acc + bb_ref[...]


def _knn(y_pad, k, d):
    sq = pl.pallas_call(
        _sq_kernel,
        grid=(NB,),
        in_specs=[pl.BlockSpec((BM, d), lambda i: (i, 0))],
        out_specs=pl.BlockSpec((BM, 1), lambda i: (i, 0)),
        out_shape=jax.ShapeDtypeStruct((NP, 1), jnp.float32),
    )(y_pad)
    sq_row = sq.reshape(1, NP)
    idxT = pl.pallas_call(
        functools.partial(_knn_kernel, k=k),
        grid=(NB,),
        in_specs=[
            pl.BlockSpec((NP, d), lambda i: (0, 0)),
            pl.BlockSpec((NP, 1), lambda i: (0, 0)),
            pl.BlockSpec((1, NP), lambda i: (0, 0)),
        ],
        out_specs=pl.BlockSpec((k, BM), lambda i: (0, i)),
        out_shape=jax.ShapeDtypeStruct((k, NP), jnp.int32),
        scratch_shapes=[pltpu.VMEM((NP, BM), jnp.float32),
                        pltpu.VMEM((CT, BM), jnp.float32),
                        pltpu.VMEM((CT, BM), jnp.int32)],
    )(y_pad, sq, sq_row)
    return idxT


def _edge(y_pad, idxT, wa, ba, wb, bb, k, d, dh, dout):
    return pl.pallas_call(
        functools.partial(_edge_mlp_kernel, k=k, d=d),
        grid=(NB,),
        in_specs=[
            pl.BlockSpec((k, BM), lambda i: (0, i),
                         memory_space=pltpu.MemorySpace.SMEM),
            pl.BlockSpec((NP, d), lambda i: (0, 0)),
            pl.BlockSpec((2 * d, dh), lambda i: (0, 0)),
            pl.BlockSpec((1, dh), lambda i: (0, 0)),
            pl.BlockSpec((dh, dout), lambda i: (0, 0)),
            pl.BlockSpec((1, dout), lambda i: (0, 0)),
        ],
        out_specs=pl.BlockSpec((BM, dout), lambda i: (i, 0)),
        out_shape=jax.ShapeDtypeStruct((NP, dout), jnp.float32),
        scratch_shapes=[pltpu.VMEM((BM, d), jnp.float32),
                        pltpu.VMEM((BM, d), jnp.float32)],
    )(idxT, y_pad, wa, ba, wb, bb)


def _layer(y_pad, wa, ba, wb, bb, k, d, dh, dout):
    idxT = _knn(y_pad, k, d)
    idx_flat = idxT.T.reshape(1, NP * k)
    half = NP // 2
    outs = []
    gs = [_sc_gather(y_pad, idx_flat[:, h * half * k:(h + 1) * half * k],
                     k, d).reshape(half, k * d) for h in range(2)]
    for h in range(2):
        outs.append(_edge_g(y_pad, gs[h], wa, ba, wb, bb, k, d, dh, dout,
                            base=h * half, nrows=half))
    return jnp.concatenate(outs, axis=0)


@jax.jit
def kernel(x, W1a, b1a, W1b, b1b, W2a, b2a, W2b, b2b):
    x_pad = jnp.pad(x, ((0, NP - N), (0, 0)))
    h = _layer(x_pad, W1a, b1a.reshape(1, -1), W1b, b1b.reshape(1, -1),
               k=16, d=128, dh=256, dout=256)
    out = _layer(h, W2a, b2a.reshape(1, -1), W2b, b2b.reshape(1, -1),
                 k=8, d=256, dh=256, dout=256)
    return out[:N]


# speedup vs baseline: 7.6055x; 1.0641x over previous
import functools

import jax
import jax.numpy as jnp
from jax.experimental import pallas as pl
from jax.experimental.pallas import tpu as pltpu
from jax.experimental.pallas import tpu_sc as plsc

N = 10000
NP = 10240
BM = 512
NB = NP // BM
WC = 64
C = NP // WC
T = 4
CT = C * T

_PREC = jax.lax.Precision.DEFAULT


def _sq_kernel(y_ref, sq_ref):
    y = y_ref[...]
    sq_ref[...] = jnp.sum(y * y, axis=1, keepdims=True)


def _knn_kernel(y_ref, sq_ref, sqr_ref, idxT_ref, dist_ref, mv_ref, gi_ref,
                *, k):
    i = pl.program_id(0)
    y = y_ref[...]
    yi = y_ref[pl.ds(i * BM, BM), :]
    dotT = jax.lax.dot_general(y, yi, (((1,), (1,)), ((), ())),
                               preferred_element_type=jnp.float32,
                               precision=_PREC)
    sqi_row = sqr_ref[:, pl.ds(i * BM, BM)]
    dist = (sqi_row - 2.0 * dotT) + sq_ref[...]
    rowg_full = jax.lax.broadcasted_iota(jnp.int32, (NP, BM), 0)
    dist_ref[...] = jnp.where(rowg_full >= N, jnp.inf, dist)

    def phase_a(c, _):
        base = c * WC
        ch = dist_ref[pl.ds(base, WC), :]
        rowg = jax.lax.broadcasted_iota(jnp.int32, (WC, BM), 0) + base
        for t in range(T):
            m = jnp.min(ch, axis=0, keepdims=True)
            g = jnp.min(jnp.where(ch == m, rowg, NP), axis=0,
                        keepdims=True)
            mv_ref[pl.ds(c * T + t, 1), :] = m
            gi_ref[pl.ds(c * T + t, 1), :] = g
            if t < T - 1:
                ch = jnp.where(rowg == g, jnp.inf, ch)
        return 0
    jax.lax.fori_loop(0, C, phase_a, 0)

    slot = jax.lax.broadcasted_iota(jnp.int32, (CT, BM), 0)
    for e in range(k):
        mv = mv_ref[...]
        v = jnp.min(mv, axis=0, keepdims=True)
        p = jnp.min(jnp.where(mv == v, slot, CT), axis=0,
                    keepdims=True)
        idx_e = jnp.min(jnp.where(slot == p, gi_ref[...], NP), axis=0,
                        keepdims=True)
        idxT_ref[pl.ds(e, 1), :] = idx_e
        mv_ref[...] = jnp.where(slot == p, jnp.inf, mv)
        if e < k - 1:
            need = (p & (T - 1)) == (T - 1)
            c_p = p >> 2
            any_need = jnp.any(need)

            @pl.when(any_need)
            def _repair(v=v, idx_e=idx_e, need=need, c_p=c_p):
                def rep(c, _):
                    base = c * WC
                    ch = dist_ref[pl.ds(base, WC), :]
                    rowg = jax.lax.broadcasted_iota(
                        jnp.int32, (WC, BM), 0) + base
                    elig = (ch > v) | ((ch == v) & (rowg > idx_e))
                    chm = jnp.where(elig, ch, jnp.inf)
                    sel = need & (c_p == c)
                    for t in range(T):
                        m = jnp.min(chm, axis=0, keepdims=True)
                        g = jnp.min(jnp.where(chm == m, rowg, NP), axis=0,
                                    keepdims=True)
                        s = c * T + t
                        mv_ref[pl.ds(s, 1), :] = jnp.where(
                            sel, m, mv_ref[pl.ds(s, 1), :])
                        gi_ref[pl.ds(s, 1), :] = jnp.where(
                            sel, g, gi_ref[pl.ds(s, 1), :])
                        if t < T - 1:
                            chm = jnp.where(rowg == g, jnp.inf, chm)
                    return 0
                jax.lax.fori_loop(0, C, rep, 0)


_GW = 128


def _sc_gather(y_pad, idx_flat, k, d):
    ni = idx_flat.shape[1]
    mesh = plsc.VectorSubcoreMesh(core_axis_name="core",
                                  subcore_axis_name="subcore")

    @pl.kernel(out_type=jax.ShapeDtypeStruct((ni, d), jnp.float32), mesh=mesh)
    def gk(y_hbm, i_hbm, o_hbm):
        def body(i_vmem, o_vmem):
            pltpu.sync_copy(y_hbm.at[i_vmem.at[0]], o_vmem)

        pltpu.emit_pipeline(
            body,
            grid=(ni // _GW,),
            in_specs=[pl.BlockSpec((1, _GW), index_map=lambda i: (0, i))],
            out_specs=[pl.BlockSpec((_GW, d), index_map=lambda i: (i, 0))],
            core_axis_name=("core", "subcore"),
            dimension_semantics=(pltpu.PARALLEL,),
        )(i_hbm, o_hbm)

    return gk(y_pad, idx_flat)


def _edge_mlp_g_kernel(g_ref, y_ref, wa_ref, ba_ref, wb_ref, bb_ref, out_ref,
                       *, k, d, base):
    i = pl.program_id(0)
    xi = y_ref[pl.ds(base + i * BM, BM), :]
    wa = wa_ref[...]
    wb = wb_ref[...]
    ba = ba_ref[...]
    acc = None
    for e in range(k):
        xj = g_ref[:, pl.ds(e * d, d)]
        feat = jnp.concatenate([xi, xj - xi], axis=1)
        h = jnp.maximum(jnp.dot(feat, wa, preferred_element_type=jnp.float32,
                                precision=_PREC) + ba, 0.0)
        pp = jnp.dot(h, wb, preferred_element_type=jnp.float32,
                     precision=_PREC)
        acc = pp if acc is None else jnp.maximum(acc, pp)
    out_ref[...] = acc + bb_ref[...]


def _edge_g(y_pad, g, wa, ba, wb, bb, k, d, dh, dout, base, nrows):
    return pl.pallas_call(
        functools.partial(_edge_mlp_g_kernel, k=k, d=d, base=base),
        grid=(nrows // BM,),
        in_specs=[
            pl.BlockSpec((BM, k * d), lambda i: (i, 0)),
            pl.BlockSpec((NP, d), lambda i: (0, 0)),
            pl.BlockSpec((2 * d, dh), lambda i: (0, 0)),
            pl.BlockSpec((1, dh), lambda i: (0, 0)),
            pl.BlockSpec((dh, dout), lambda i: (0, 0)),
            pl.BlockSpec((1, dout), lambda i: (0, 0)),
        ],
        out_specs=pl.BlockSpec((BM, dout), lambda i: (i, 0)),
        out_shape=jax.ShapeDtypeStruct((nrows, dout), jnp.float32),
    )(g, y_pad, wa, ba, wb, bb)


def _edge_mlp_kernel(idxT_ref, y_ref, wa_ref, ba_ref, wb_ref, bb_ref, out_ref,
                     ga_ref, gb_ref, *, k, d):
    i = pl.program_id(0)
    xi = y_ref[pl.ds(i * BM, BM), :]
    wa = wa_ref[...]
    wb = wb_ref[...]
    ba = ba_ref[...]
    acc = None
    bufs = (ga_ref, gb_ref)
    for e in range(k):
        gr = bufs[e & 1]

        def body(r8, _, gr=gr, e=e):
            b = r8 * 8
            for u in range(8):
                j = idxT_ref[e, b + u]
                gr[pl.ds(b + u, 1), :] = y_ref[pl.ds(j, 1), :]
            return 0
        jax.lax.fori_loop(0, BM // 8, body, 0)
        feat = jnp.concatenate([xi, gr[...] - xi], axis=1)
        h = jnp.maximum(jnp.dot(feat, wa, preferred_element_type=jnp.float32,
                                precision=_PREC) + ba, 0.0)
        pp = jnp.dot(h, wb, preferred_element_type=jnp.float32,
                     precision=_PREC)
        acc = pp if acc is None else jnp.maximum(acc, pp)
    out_ref[...] = acc + bb_ref[...]


def _knn(y_pad, k, d):
    sq = pl.pallas_call(
        _sq_kernel,
        grid=(NB,),
        in_specs=[pl.BlockSpec((BM, d), lambda i: (i, 0))],
        out_specs=pl.BlockSpec((BM, 1), lambda i: (i, 0)),
        out_shape=jax.ShapeDtypeStruct((NP, 1), jnp.float32),
    )(y_pad)
    sq_row = sq.reshape(1, NP)
    idxT = pl.pallas_call(
        functools.partial(_knn_kernel, k=k),
        grid=(NB,),
        in_specs=[
            pl.BlockSpec((NP, d), lambda i: (0, 0)),
            pl.BlockSpec((NP, 1), lambda i: (0, 0)),
            pl.BlockSpec((1, NP), lambda i: (0, 0)),
        ],
        out_specs=pl.BlockSpec((k, BM), lambda i: (0, i)),
        out_shape=jax.ShapeDtypeStruct((k, NP), jnp.int32),
        scratch_shapes=[pltpu.VMEM((NP, BM), jnp.float32),
                        pltpu.VMEM((CT, BM), jnp.float32),
                        pltpu.VMEM((CT, BM), jnp.int32)],
    )(y_pad, sq, sq_row)
    return idxT


def _edge(y_pad, idxT, wa, ba, wb, bb, k, d, dh, dout):
    return pl.pallas_call(
        functools.partial(_edge_mlp_kernel, k=k, d=d),
        grid=(NB,),
        in_specs=[
            pl.BlockSpec((k, BM), lambda i: (0, i),
                         memory_space=pltpu.MemorySpace.SMEM),
            pl.BlockSpec((NP, d), lambda i: (0, 0)),
            pl.BlockSpec((2 * d, dh), lambda i: (0, 0)),
            pl.BlockSpec((1, dh), lambda i: (0, 0)),
            pl.BlockSpec((dh, dout), lambda i: (0, 0)),
            pl.BlockSpec((1, dout), lambda i: (0, 0)),
        ],
        out_specs=pl.BlockSpec((BM, dout), lambda i: (i, 0)),
        out_shape=jax.ShapeDtypeStruct((NP, dout), jnp.float32),
        scratch_shapes=[pltpu.VMEM((BM, d), jnp.float32),
                        pltpu.VMEM((BM, d), jnp.float32)],
    )(idxT, y_pad, wa, ba, wb, bb)


def _layer(y_pad, wa, ba, wb, bb, k, d, dh, dout):
    idxT = _knn(y_pad, k, d)
    idx_flat = idxT.T.reshape(1, NP * k)
    half = NP // 2
    outs = []
    gs = [_sc_gather(y_pad, idx_flat[:, h * half * k:(h + 1) * half * k],
                     k, d).reshape(half, k * d) for h in range(2)]
    for h in range(2):
        outs.append(_edge_g(y_pad, gs[h], wa, ba, wb, bb, k, d, dh, dout,
                            base=h * half, nrows=half))
    return jnp.concatenate(outs, axis=0)


@jax.jit
def kernel(x, W1a, b1a, W1b, b1b, W2a, b2a, W2b, b2b):
    x_pad = jnp.pad(x, ((0, NP - N), (0, 0)))
    h = _layer(x_pad, W1a, b1a.reshape(1, -1), W1b, b1b.reshape(1, -1),
               k=16, d=128, dh=256, dout=256)
    out = _layer(h, W2a, b2a.reshape(1, -1), W2b, b2b.reshape(1, -1),
                 k=8, d=256, dh=256, dout=256)
    return out[:N]


# speedup vs baseline: 7.8856x; 1.0368x over previous
import functools

import jax
import jax.numpy as jnp
from jax.experimental import pallas as pl
from jax.experimental.pallas import tpu as pltpu
from jax.experimental.pallas import tpu_sc as plsc

N = 10000
NP = 10240
BM = 512
NB = NP // BM
WC = 64
C = NP // WC
T = 4
CT = C * T

_PREC = jax.lax.Precision.DEFAULT


def _sq_kernel(y_ref, sq_ref):
    y = y_ref[...]
    sq_ref[...] = jnp.sum(y * y, axis=1, keepdims=True)


def _knn_kernel(y_ref, sq_ref, sqr_ref, idxT_ref, dist_ref, mv_ref, gi_ref,
                *, k, base):
    i = pl.program_id(0)
    y = y_ref[...]
    yi = y_ref[pl.ds(base + i * BM, BM), :]
    dotT = jax.lax.dot_general(y, yi, (((1,), (1,)), ((), ())),
                               preferred_element_type=jnp.float32,
                               precision=_PREC)
    sqi_row = sqr_ref[:, pl.ds(base + i * BM, BM)]
    dist = (sqi_row - 2.0 * dotT) + sq_ref[...]
    rowg_full = jax.lax.broadcasted_iota(jnp.int32, (NP, BM), 0)
    dist_ref[...] = jnp.where(rowg_full >= N, jnp.inf, dist)

    def phase_a(c, _):
        base = c * WC
        ch = dist_ref[pl.ds(base, WC), :]
        rowg = jax.lax.broadcasted_iota(jnp.int32, (WC, BM), 0) + base
        for t in range(T):
            m = jnp.min(ch, axis=0, keepdims=True)
            g = jnp.min(jnp.where(ch == m, rowg, NP), axis=0,
                        keepdims=True)
            mv_ref[pl.ds(c * T + t, 1), :] = m
            gi_ref[pl.ds(c * T + t, 1), :] = g
            if t < T - 1:
                ch = jnp.where(rowg == g, jnp.inf, ch)
        return 0
    jax.lax.fori_loop(0, C, phase_a, 0)

    slot = jax.lax.broadcasted_iota(jnp.int32, (CT, BM), 0)
    for e in range(k):
        mv = mv_ref[...]
        v = jnp.min(mv, axis=0, keepdims=True)
        p = jnp.min(jnp.where(mv == v, slot, CT), axis=0,
                    keepdims=True)
        idx_e = jnp.min(jnp.where(slot == p, gi_ref[...], NP), axis=0,
                        keepdims=True)
        idxT_ref[pl.ds(e, 1), :] = idx_e
        mv_ref[...] = jnp.where(slot == p, jnp.inf, mv)
        if e < k - 1:
            need = (p & (T - 1)) == (T - 1)
            c_p = p >> 2
            any_need = jnp.any(need)

            @pl.when(any_need)
            def _repair(v=v, idx_e=idx_e, need=need, c_p=c_p):
                def rep(c, _):
                    base = c * WC
                    ch = dist_ref[pl.ds(base, WC), :]
                    rowg = jax.lax.broadcasted_iota(
                        jnp.int32, (WC, BM), 0) + base
                    elig = (ch > v) | ((ch == v) & (rowg > idx_e))
                    chm = jnp.where(elig, ch, jnp.inf)
                    sel = need & (c_p == c)
                    for t in range(T):
                        m = jnp.min(chm, axis=0, keepdims=True)
                        g = jnp.min(jnp.where(chm == m, rowg, NP), axis=0,
                                    keepdims=True)
                        s = c * T + t
                        mv_ref[pl.ds(s, 1), :] = jnp.where(
                            sel, m, mv_ref[pl.ds(s, 1), :])
                        gi_ref[pl.ds(s, 1), :] = jnp.where(
                            sel, g, gi_ref[pl.ds(s, 1), :])
                        if t < T - 1:
                            chm = jnp.where(rowg == g, jnp.inf, chm)
                    return 0
                jax.lax.fori_loop(0, C, rep, 0)


_GW = 128


def _sc_gather(y_pad, idx_flat, k, d):
    ni = idx_flat.shape[1]
    mesh = plsc.VectorSubcoreMesh(core_axis_name="core",
                                  subcore_axis_name="subcore")

    @pl.kernel(out_type=jax.ShapeDtypeStruct((ni, d), jnp.float32), mesh=mesh)
    def gk(y_hbm, i_hbm, o_hbm):
        def body(i_vmem, o_vmem):
            pltpu.sync_copy(y_hbm.at[i_vmem.at[0]], o_vmem)

        pltpu.emit_pipeline(
            body,
            grid=(ni // _GW,),
            in_specs=[pl.BlockSpec((1, _GW), index_map=lambda i: (0, i))],
            out_specs=[pl.BlockSpec((_GW, d), index_map=lambda i: (i, 0))],
            core_axis_name=("core", "subcore"),
            dimension_semantics=(pltpu.PARALLEL,),
        )(i_hbm, o_hbm)

    return gk(y_pad, idx_flat)


def _edge_mlp_g_kernel(g_ref, y_ref, wa_ref, ba_ref, wb_ref, bb_ref, out_ref,
                       *, k, d, base):
    i = pl.program_id(0)
    xi = y_ref[pl.ds(base + i * BM, BM), :]
    wa = wa_ref[...]
    wb = wb_ref[...]
    ba = ba_ref[...]
    acc = None
    for e in range(k):
        xj = g_ref[:, pl.ds(e * d, d)]
        feat = jnp.concatenate([xi, xj - xi], axis=1)
        h = jnp.maximum(jnp.dot(feat, wa, preferred_element_type=jnp.float32,
                                precision=_PREC) + ba, 0.0)
        pp = jnp.dot(h, wb, preferred_element_type=jnp.float32,
                     precision=_PREC)
        acc = pp if acc is None else jnp.maximum(acc, pp)
    out_ref[...] = acc + bb_ref[...]


def _edge_g(y_pad, g, wa, ba, wb, bb, k, d, dh, dout, base, nrows):
    return pl.pallas_call(
        functools.partial(_edge_mlp_g_kernel, k=k, d=d, base=base),
        grid=(nrows // BM,),
        in_specs=[
            pl.BlockSpec((BM, k * d), lambda i: (i, 0)),
            pl.BlockSpec((NP, d), lambda i: (0, 0)),
            pl.BlockSpec((2 * d, dh), lambda i: (0, 0)),
            pl.BlockSpec((1, dh), lambda i: (0, 0)),
            pl.BlockSpec((dh, dout), lambda i: (0, 0)),
            pl.BlockSpec((1, dout), lambda i: (0, 0)),
        ],
        out_specs=pl.BlockSpec((BM, dout), lambda i: (i, 0)),
        out_shape=jax.ShapeDtypeStruct((nrows, dout), jnp.float32),
    )(g, y_pad, wa, ba, wb, bb)


def _edge_mlp_kernel(idxT_ref, y_ref, wa_ref, ba_ref, wb_ref, bb_ref, out_ref,
                     ga_ref, gb_ref, *, k, d):
    i = pl.program_id(0)
    xi = y_ref[pl.ds(i * BM, BM), :]
    wa = wa_ref[...]
    wb = wb_ref[...]
    ba = ba_ref[...]
    acc = None
    bufs = (ga_ref, gb_ref)
    for e in range(k):
        gr = bufs[e & 1]

        def body(r8, _, gr=gr, e=e):
            b = r8 * 8
            for u in range(8):
                j = idxT_ref[e, b + u]
                gr[pl.ds(b + u, 1), :] = y_ref[pl.ds(j, 1), :]
            return 0
        jax.lax.fori_loop(0, BM // 8, body, 0)
        feat = jnp.concatenate([xi, gr[...] - xi], axis=1)
        h = jnp.maximum(jnp.dot(feat, wa, preferred_element_type=jnp.float32,
                                precision=_PREC) + ba, 0.0)
        pp = jnp.dot(h, wb, preferred_element_type=jnp.float32,
                     precision=_PREC)
        acc = pp if acc is None else jnp.maximum(acc, pp)
    out_ref[...] = acc + bb_ref[...]


def _sq(y_pad, d):
    return pl.pallas_call(
        _sq_kernel,
        grid=(NB,),
        in_specs=[pl.BlockSpec((BM, d), lambda i: (i, 0))],
        out_specs=pl.BlockSpec((BM, 1), lambda i: (i, 0)),
        out_shape=jax.ShapeDtypeStruct((NP, 1), jnp.float32),
    )(y_pad)


def _knn(y_pad, sq, sq_row, k, d, base, ncols):
    return pl.pallas_call(
        functools.partial(_knn_kernel, k=k, base=base),
        grid=(ncols // BM,),
        in_specs=[
            pl.BlockSpec((NP, d), lambda i: (0, 0)),
            pl.BlockSpec((NP, 1), lambda i: (0, 0)),
            pl.BlockSpec((1, NP), lambda i: (0, 0)),
        ],
        out_specs=pl.BlockSpec((k, BM), lambda i: (0, i)),
        out_shape=jax.ShapeDtypeStruct((k, ncols), jnp.int32),
        scratch_shapes=[pltpu.VMEM((NP, BM), jnp.float32),
                        pltpu.VMEM((CT, BM), jnp.float32),
                        pltpu.VMEM((CT, BM), jnp.int32)],
    )(y_pad, sq, sq_row)


def _edge(y_pad, idxT, wa, ba, wb, bb, k, d, dh, dout):
    return pl.pallas_call(
        functools.partial(_edge_mlp_kernel, k=k, d=d),
        grid=(NB,),
        in_specs=[
            pl.BlockSpec((k, BM), lambda i: (0, i),
                         memory_space=pltpu.MemorySpace.SMEM),
            pl.BlockSpec((NP, d), lambda i: (0, 0)),
            pl.BlockSpec((2 * d, dh), lambda i: (0, 0)),
            pl.BlockSpec((1, dh), lambda i: (0, 0)),
            pl.BlockSpec((dh, dout), lambda i: (0, 0)),
            pl.BlockSpec((1, dout), lambda i: (0, 0)),
        ],
        out_specs=pl.BlockSpec((BM, dout), lambda i: (i, 0)),
        out_shape=jax.ShapeDtypeStruct((NP, dout), jnp.float32),
        scratch_shapes=[pltpu.VMEM((BM, d), jnp.float32),
                        pltpu.VMEM((BM, d), jnp.float32)],
    )(idxT, y_pad, wa, ba, wb, bb)


def _layer(y_pad, wa, ba, wb, bb, k, d, dh, dout):
    half = NP // 2
    sq = _sq(y_pad, d)
    sq_row = sq.reshape(1, NP)
    idxs = [_knn(y_pad, sq, sq_row, k, d, base=h * half, ncols=half)
            for h in range(2)]
    gs = [_sc_gather(y_pad, idxs[h].T.reshape(1, half * k), k, d)
          .reshape(half, k * d) for h in range(2)]
    outs = [_edge_g(y_pad, gs[h], wa, ba, wb, bb, k, d, dh, dout,
                    base=h * half, nrows=half) for h in range(2)]
    return jnp.concatenate(outs, axis=0)


@jax.jit
def kernel(x, W1a, b1a, W1b, b1b, W2a, b2a, W2b, b2b):
    x_pad = jnp.pad(x, ((0, NP - N), (0, 0)))
    h = _layer(x_pad, W1a, b1a.reshape(1, -1), W1b, b1b.reshape(1, -1),
               k=16, d=128, dh=256, dout=256)
    out = _layer(h, W2a, b2a.reshape(1, -1), W2b, b2b.reshape(1, -1),
                 k=8, d=256, dh=256, dout=256)
    return out[:N]
